# Initial kernel scaffold; baseline (speedup 1.0000x reference)
import jax
import jax.numpy as jnp
from jax.experimental import pallas as pl


def kernel(features, edge_index, labels, W0, b0, Wp, bp):
    raise NotImplementedError("write your pallas kernel here")


# speedup vs baseline: 2.3864x; 2.3864x over previous
import functools

import jax
import jax.numpy as jnp
from jax import lax
from jax.experimental import pallas as pl
from jax.experimental.pallas import tpu as pltpu
from jax.experimental.pallas import tpu_sc as plsc

_N = 10000
_E = 320000
_D = 128

_NS = 16
_CHUNK = 128
_CPW = 158
_EPW = _CPW * _CHUNK
_EPAD = _NS * _EPW
_NPAD = 10240
_RPT = _NPAD // _NS
_BM = 2048

_mesh = plsc.VectorSubcoreMesh(
    core_axis_name="c", subcore_axis_name="s", num_cores=1)


_DROWS = _NPAD // 16


@functools.partial(
    pl.kernel,
    out_type=jax.ShapeDtypeStruct((_DROWS, _D), jnp.float32),
    mesh=_mesh,
    scratch_types=[
        pltpu.VMEM((_EPW,), jnp.int32),
        pltpu.VMEM((1, _CHUNK), jnp.int32),
        pltpu.VMEM((1, _CHUNK), jnp.int32),
        pltpu.VMEM((_CHUNK, _D), jnp.float32),
        pltpu.VMEM_SHARED((_DROWS, _D), jnp.float32),
        pltpu.SemaphoreType.DMA,
    ],
)
def _sc_deg(dst_hbm, eye_hbm, zeros_hbm, out_hbm, dstv, rowv, colv, vals,
            shdeg, sem):
    sid = lax.axis_index("s")
    rpt = _DROWS // _NS
    pltpu.sync_copy(dst_hbm.at[sid], dstv)
    pltpu.sync_copy(zeros_hbm.at[pl.ds(0, rpt)],
                    shdeg.at[pl.ds(sid * rpt, rpt)])
    plsc.subcore_barrier()

    def body(j, carry):
        for k in range(_CHUNK // 16):
            d16 = dstv[pl.ds(j * _CHUNK + k * 16, 16)]
            rowv[0, pl.ds(k * 16, 16)] = lax.shift_right_logical(d16, 4)
            colv[0, pl.ds(k * 16, 16)] = lax.bitwise_and(d16, 15)
        pltpu.async_copy(eye_hbm.at[colv.at[0]], vals, sem).wait()
        pltpu.sync_copy(vals, shdeg.at[rowv.at[0]], add=True)
        return carry

    lax.fori_loop(0, _CPW, body, 0)
    plsc.subcore_barrier()
    pltpu.sync_copy(shdeg.at[pl.ds(sid * rpt, rpt)],
                    out_hbm.at[pl.ds(sid * rpt, rpt)])


_HALF = _NPAD // 2
_AROWS = _HALF + _CHUNK
_ZPT = _AROWS // _NS
_OPT = _HALF // _NS


@functools.partial(
    pl.kernel,
    out_type=jax.ShapeDtypeStruct((_NPAD, _D), jnp.float32),
    mesh=_mesh,
    scratch_types=[
        pltpu.VMEM((_CPW, _CHUNK), jnp.int32),
        pltpu.VMEM((_EPW,), jnp.int32),
        pltpu.VMEM((1, _CHUNK), jnp.int32),
        pltpu.VMEM((_CHUNK, _D), jnp.float32),
        pltpu.VMEM((_CHUNK, _D), jnp.float32),
        pltpu.VMEM_SHARED((_AROWS, _D), jnp.float32),
        pltpu.SemaphoreType.DMA,
    ],
)
def _sc_agg(xws_hbm, src_hbm, dst_hbm, zeros_hbm, out_hbm,
            srcv, dstv, rbuf, rows, zbuf, shagg, sem):
    sid = lax.axis_index("s")
    pltpu.sync_copy(src_hbm.at[sid], srcv)
    pltpu.sync_copy(dst_hbm.at[sid], dstv)
    pltpu.sync_copy(zeros_hbm, zbuf)

    def zero_acc():
        zb = sid * _ZPT
        off = 0
        while off < _ZPT:
            n = min(_CHUNK, _ZPT - off)
            pltpu.sync_copy(zbuf.at[pl.ds(0, n)],
                            shagg.at[pl.ds(zb + off, n)])
            off += n

    for p in range(2):
        zero_acc()
        plsc.subcore_barrier()
        lo = p * _HALF

        def body(j, carry):
            for k in range(_CHUNK // 16):
                d16 = dstv[pl.ds(j * _CHUNK + k * 16, 16)]
                t16 = d16 - lo
                ok = jnp.logical_and(t16 >= 0, t16 < _HALF)
                rbuf[0, pl.ds(k * 16, 16)] = jnp.where(ok, t16, _HALF)
            pltpu.async_copy(xws_hbm.at[srcv.at[j]], rows, sem).wait()
            pltpu.sync_copy(rows, shagg.at[rbuf.at[0]], add=True)
            return carry

        lax.fori_loop(0, _CPW, body, 0)
        plsc.subcore_barrier()
        pltpu.sync_copy(shagg.at[pl.ds(sid * _OPT, _OPT)],
                        out_hbm.at[pl.ds(lo + sid * _OPT, _OPT)])
        plsc.subcore_barrier()


def _mm_body(x_ref, w_ref, o_ref):
    o_ref[...] = jnp.dot(x_ref[...], w_ref[...],
                         preferred_element_type=jnp.float32)


def _scale_body(xw_ref, deg_ref, xws_ref, norm_ref):
    norm = lax.rsqrt(jnp.maximum(deg_ref[...], 1.0))
    norm_ref[...] = norm
    xws_ref[...] = xw_ref[...] * norm


def _loss_body(a_ref, norm_ref, b0_ref, wp_ref, bp_ref, o_ref):
    j = pl.program_id(0)
    h = jnp.maximum(a_ref[...] * norm_ref[...] + b0_ref[...], 0.0)
    v = jnp.sum(wp_ref[...], axis=1, keepdims=True)
    c = jnp.sum(bp_ref[...])
    s = jnp.dot(h, v, preferred_element_type=jnp.float32) + c
    row = lax.broadcasted_iota(jnp.int32, (_BM, 1), 0) + j * _BM
    t = jnp.abs(s)
    term = t + 2.0 * jnp.log1p(jnp.exp(-t))
    term = jnp.where(row < _N, term, 0.0)
    part = (jnp.sum(term) / (2.0 * _N)).reshape(1, 1)

    @pl.when(j == 0)
    def _():
        o_ref[...] = jnp.zeros_like(part)

    o_ref[...] += part


def _tc_matmul(x, w):
    return pl.pallas_call(
        _mm_body,
        grid=(_NPAD // _BM,),
        in_specs=[pl.BlockSpec((_BM, _D), lambda i: (i, 0)),
                  pl.BlockSpec((_D, _D), lambda i: (0, 0))],
        out_specs=pl.BlockSpec((_BM, _D), lambda i: (i, 0)),
        out_shape=jax.ShapeDtypeStruct((_NPAD, _D), jnp.float32),
    )(x, w)


def _tc_scale(xw, deg):
    return pl.pallas_call(
        _scale_body,
        grid=(_NPAD // _BM,),
        in_specs=[pl.BlockSpec((_BM, _D), lambda i: (i, 0)),
                  pl.BlockSpec((_BM, 1), lambda i: (i, 0))],
        out_specs=[pl.BlockSpec((_BM, _D), lambda i: (i, 0)),
                   pl.BlockSpec((_BM, 1), lambda i: (i, 0))],
        out_shape=[jax.ShapeDtypeStruct((_NPAD, _D), jnp.float32),
                   jax.ShapeDtypeStruct((_NPAD, 1), jnp.float32)],
    )(xw, deg)


def _tc_loss(a, norm, b0r, wp, bpr):
    return pl.pallas_call(
        _loss_body,
        grid=(_NPAD // _BM,),
        in_specs=[pl.BlockSpec((_BM, _D), lambda i: (i, 0)),
                  pl.BlockSpec((_BM, 1), lambda i: (i, 0)),
                  pl.BlockSpec((1, _D), lambda i: (0, 0)),
                  pl.BlockSpec((_D, _D), lambda i: (0, 0)),
                  pl.BlockSpec((1, _D), lambda i: (0, 0))],
        out_specs=pl.BlockSpec((1, 1), lambda i: (0, 0)),
        out_shape=jax.ShapeDtypeStruct((1, 1), jnp.float32),
    )(a, norm, b0r, wp, bpr)


def kernel(features, edge_index, labels, W0, b0, Wp, bp):
    del labels
    xpad = jnp.pad(features, ((0, _NPAD - _N), (0, 0)))
    pad = _EPAD - _E
    src_p = jnp.concatenate(
        [edge_index[0], jnp.zeros((pad,), jnp.int32)]).reshape(_NS, _CPW, _CHUNK)
    dst_flat = jnp.concatenate(
        [edge_index[1], jnp.full((pad,), _N, jnp.int32)]).reshape(_NS, _EPW)
    zeros128 = jnp.zeros((_CHUNK, _D), jnp.float32)

    eye_w = jnp.eye(16, _D, dtype=jnp.float32)
    deg = _sc_deg(dst_flat, eye_w, zeros128)
    xw = _tc_matmul(xpad, W0)
    degc = deg[:, :16].reshape(_NPAD, 1)
    xws, norm = _tc_scale(xw, degc)
    agg = _sc_agg(xws, src_p, dst_flat, zeros128)
    loss = _tc_loss(agg, norm,
                    b0.reshape(1, _D), Wp, bp.reshape(1, _D))
    return loss[0, 0]


# speedup vs baseline: 4.8083x; 2.0149x over previous
import functools

import jax
import jax.numpy as jnp
from jax import lax
from jax.experimental import pallas as pl
from jax.experimental.pallas import tpu as pltpu
from jax.experimental.pallas import tpu_sc as plsc

_N = 10000
_E = 320000
_D = 128

_NS = 16
_CHUNK = 128
_CPW = 158
_EPW = _CPW * _CHUNK
_EPAD = _NS * _EPW
_NPAD = 10240
_RPT = _NPAD // _NS
_BM = 2048

_mesh = plsc.VectorSubcoreMesh(
    core_axis_name="c", subcore_axis_name="s", num_cores=1)


@functools.partial(
    pl.kernel,
    out_type=jax.ShapeDtypeStruct((_NS, _NPAD), jnp.float32),
    mesh=_mesh,
    scratch_types=[
        pltpu.VMEM((_EPW,), jnp.int32),
        pltpu.VMEM((_NPAD,), jnp.float32),
    ],
    compiler_params=pltpu.CompilerParams(needs_layout_passes=False),
)
def _sc_deg(dst_hbm, zeros_hbm, out_hbm, dstv, hist):
    sid = lax.axis_index("s")
    pltpu.sync_copy(dst_hbm.at[sid], dstv)
    pltpu.sync_copy(zeros_hbm, hist)
    ones = jnp.full((16,), 1.0, jnp.float32)

    def body(i, carry):
        idx = dstv[pl.ds(i * 16, 16)]
        plsc.addupdate_scatter(hist, [idx], ones)
        return carry

    lax.fori_loop(0, _EPW // 16, body, 0)
    pltpu.sync_copy(hist, out_hbm.at[sid])


_HALF = _NPAD // 2
_AROWS = _HALF + _CHUNK
_ZPT = _AROWS // _NS
_OPT = _HALF // _NS


@functools.partial(
    pl.kernel,
    out_type=jax.ShapeDtypeStruct((_NPAD, _D), jnp.float32),
    mesh=_mesh,
    scratch_types=[
        pltpu.VMEM((_CPW, _CHUNK), jnp.int32),
        pltpu.VMEM((_EPW,), jnp.int32),
        pltpu.VMEM((1, _CHUNK), jnp.int32),
        pltpu.VMEM((_CHUNK, _D), jnp.float32),
        pltpu.VMEM((_CHUNK, _D), jnp.float32),
        pltpu.VMEM_SHARED((_AROWS, _D), jnp.float32),
        pltpu.SemaphoreType.DMA,
    ],
)
def _sc_agg(xws_hbm, src_hbm, dst_hbm, zeros_hbm, out_hbm,
            srcv, dstv, rbuf, rows, zbuf, shagg, sem):
    sid = lax.axis_index("s")
    pltpu.sync_copy(src_hbm.at[sid], srcv)
    pltpu.sync_copy(dst_hbm.at[sid], dstv)
    pltpu.sync_copy(zeros_hbm, zbuf)

    def zero_acc():
        zb = sid * _ZPT
        off = 0
        while off < _ZPT:
            n = min(_CHUNK, _ZPT - off)
            pltpu.sync_copy(zbuf.at[pl.ds(0, n)],
                            shagg.at[pl.ds(zb + off, n)])
            off += n

    for p in range(2):
        zero_acc()
        plsc.subcore_barrier()
        lo = p * _HALF

        def body(j, carry):
            for k in range(_CHUNK // 16):
                d16 = dstv[pl.ds(j * _CHUNK + k * 16, 16)]
                t16 = d16 - lo
                ok = jnp.logical_and(t16 >= 0, t16 < _HALF)
                rbuf[0, pl.ds(k * 16, 16)] = jnp.where(ok, t16, _HALF)
            pltpu.async_copy(xws_hbm.at[srcv.at[j]], rows, sem).wait()
            pltpu.sync_copy(rows, shagg.at[rbuf.at[0]], add=True)
            return carry

        lax.fori_loop(0, _CPW, body, 0)
        plsc.subcore_barrier()
        pltpu.sync_copy(shagg.at[pl.ds(sid * _OPT, _OPT)],
                        out_hbm.at[pl.ds(lo + sid * _OPT, _OPT)])
        plsc.subcore_barrier()


def _mm_body(x_ref, w_ref, o_ref):
    o_ref[...] = jnp.dot(x_ref[...], w_ref[...],
                         preferred_element_type=jnp.float32)


def _scale_body(xw_ref, hist_ref, xws_ref, norm_ref):
    ones = jnp.ones((_NS, 1), jnp.float32)
    deg = lax.dot_general(hist_ref[...], ones, (((0,), (0,)), ((), ())),
                          preferred_element_type=jnp.float32)
    norm = lax.rsqrt(jnp.maximum(deg, 1.0))
    norm_ref[...] = norm
    xws_ref[...] = xw_ref[...] * norm


def _loss_body(a_ref, norm_ref, b0_ref, wp_ref, bp_ref, o_ref):
    j = pl.program_id(0)
    h = jnp.maximum(a_ref[...] * norm_ref[...] + b0_ref[...], 0.0)
    v = jnp.sum(wp_ref[...], axis=1, keepdims=True)
    c = jnp.sum(bp_ref[...])
    s = jnp.dot(h, v, preferred_element_type=jnp.float32) + c
    row = lax.broadcasted_iota(jnp.int32, (_BM, 1), 0) + j * _BM
    t = jnp.abs(s)
    term = t + 2.0 * jnp.log1p(jnp.exp(-t))
    term = jnp.where(row < _N, term, 0.0)
    part = (jnp.sum(term) / (2.0 * _N)).reshape(1, 1)

    @pl.when(j == 0)
    def _():
        o_ref[...] = jnp.zeros_like(part)

    o_ref[...] += part


def _tc_matmul(x, w):
    return pl.pallas_call(
        _mm_body,
        grid=(_NPAD // _BM,),
        in_specs=[pl.BlockSpec((_BM, _D), lambda i: (i, 0)),
                  pl.BlockSpec((_D, _D), lambda i: (0, 0))],
        out_specs=pl.BlockSpec((_BM, _D), lambda i: (i, 0)),
        out_shape=jax.ShapeDtypeStruct((_NPAD, _D), jnp.float32),
    )(x, w)


def _tc_scale(xw, hist):
    return pl.pallas_call(
        _scale_body,
        grid=(_NPAD // _BM,),
        in_specs=[pl.BlockSpec((_BM, _D), lambda i: (i, 0)),
                  pl.BlockSpec((_NS, _BM), lambda i: (0, i))],
        out_specs=[pl.BlockSpec((_BM, _D), lambda i: (i, 0)),
                   pl.BlockSpec((_BM, 1), lambda i: (i, 0))],
        out_shape=[jax.ShapeDtypeStruct((_NPAD, _D), jnp.float32),
                   jax.ShapeDtypeStruct((_NPAD, 1), jnp.float32)],
    )(xw, hist)


def _tc_loss(a, norm, b0r, wp, bpr):
    return pl.pallas_call(
        _loss_body,
        grid=(_NPAD // _BM,),
        in_specs=[pl.BlockSpec((_BM, _D), lambda i: (i, 0)),
                  pl.BlockSpec((_BM, 1), lambda i: (i, 0)),
                  pl.BlockSpec((1, _D), lambda i: (0, 0)),
                  pl.BlockSpec((_D, _D), lambda i: (0, 0)),
                  pl.BlockSpec((1, _D), lambda i: (0, 0))],
        out_specs=pl.BlockSpec((1, 1), lambda i: (0, 0)),
        out_shape=jax.ShapeDtypeStruct((1, 1), jnp.float32),
    )(a, norm, b0r, wp, bpr)


def kernel(features, edge_index, labels, W0, b0, Wp, bp):
    del labels
    xpad = jnp.pad(features, ((0, _NPAD - _N), (0, 0)))
    pad = _EPAD - _E
    src_p = jnp.concatenate(
        [edge_index[0], jnp.zeros((pad,), jnp.int32)]).reshape(_NS, _CPW, _CHUNK)
    dst_flat = jnp.concatenate(
        [edge_index[1], jnp.full((pad,), _N, jnp.int32)]).reshape(_NS, _EPW)
    zeros128 = jnp.zeros((_CHUNK, _D), jnp.float32)

    zeros_n = jnp.zeros((_NPAD,), jnp.float32)
    hist = _sc_deg(dst_flat, zeros_n)
    xw = _tc_matmul(xpad, W0)
    xws, norm = _tc_scale(xw, hist)
    agg = _sc_agg(xws, src_p, dst_flat, zeros128)
    loss = _tc_loss(agg, norm,
                    b0.reshape(1, _D), Wp, bp.reshape(1, _D))
    return loss[0, 0]


# speedup vs baseline: 6.4204x; 1.3353x over previous
import functools

import jax
import jax.numpy as jnp
from jax import lax
from jax.experimental import pallas as pl
from jax.experimental.pallas import tpu as pltpu
from jax.experimental.pallas import tpu_sc as plsc

_N = 10000
_E = 320000
_D = 128

_NS = 16
_CHUNK = 128
_CPW = 158
_EPW = _CPW * _CHUNK
_EPAD = _NS * _EPW
_NPAD = 10240
_RPT = _NPAD // _NS
_BM = 2048

_mesh = plsc.VectorSubcoreMesh(
    core_axis_name="c", subcore_axis_name="s", num_cores=1)


@functools.partial(
    pl.kernel,
    out_type=jax.ShapeDtypeStruct((_NS, _NPAD), jnp.float32),
    mesh=_mesh,
    scratch_types=[
        pltpu.VMEM((_EPW,), jnp.int32),
        pltpu.VMEM((_NPAD,), jnp.float32),
    ],
    compiler_params=pltpu.CompilerParams(needs_layout_passes=False),
)
def _sc_deg(dst_hbm, zeros_hbm, out_hbm, dstv, hist):
    sid = lax.axis_index("s")
    pltpu.sync_copy(dst_hbm.at[sid], dstv)
    pltpu.sync_copy(zeros_hbm, hist)
    ones = jnp.full((16,), 1.0, jnp.float32)

    def body(i, carry):
        idx = dstv[pl.ds(i * 16, 16)]
        plsc.addupdate_scatter(hist, [idx], ones)
        return carry

    lax.fori_loop(0, _EPW // 16, body, 0)
    pltpu.sync_copy(hist, out_hbm.at[sid])


_NC = 2
_CPW2 = 79
_EPW2 = _CPW2 * _CHUNK
_EPAD2 = _NC * _NS * _EPW2
_HALF = _NPAD // 2
_AROWS = _HALF + _CHUNK
_ZPT = _AROWS // _NS
_OPT = _HALF // _NS

_mesh2 = plsc.VectorSubcoreMesh(
    core_axis_name="c", subcore_axis_name="s", num_cores=_NC)


@functools.partial(
    pl.kernel,
    out_type=jax.ShapeDtypeStruct((_NC, _NPAD, _D), jnp.float32),
    mesh=_mesh2,
    scratch_types=[
        pltpu.VMEM((_CPW2, _CHUNK), jnp.int32),
        pltpu.VMEM((_EPW2,), jnp.int32),
        pltpu.VMEM((1, _CHUNK), jnp.int32),
        pltpu.VMEM((_CHUNK, _D), jnp.float32),
        pltpu.VMEM((_CHUNK, _D), jnp.float32),
        pltpu.VMEM_SHARED((_AROWS, _D), jnp.float32),
        pltpu.SemaphoreType.DMA,
    ],
)
def _sc_agg(xws_hbm, src_hbm, dst_hbm, zeros_hbm, out_hbm,
            srcv, dstv, rbuf, rows, zbuf, shagg, sem):
    cid = lax.axis_index("c")
    sid = lax.axis_index("s")
    pltpu.sync_copy(src_hbm.at[cid, sid], srcv)
    pltpu.sync_copy(dst_hbm.at[cid, sid], dstv)
    pltpu.sync_copy(zeros_hbm, zbuf)

    def zero_acc():
        zb = sid * _ZPT
        off = 0
        while off < _ZPT:
            n = min(_CHUNK, _ZPT - off)
            pltpu.sync_copy(zbuf.at[pl.ds(0, n)],
                            shagg.at[pl.ds(zb + off, n)])
            off += n

    for p in range(2):
        zero_acc()
        plsc.subcore_barrier()
        lo = p * _HALF

        def body(j, carry):
            for k in range(_CHUNK // 16):
                d16 = dstv[pl.ds(j * _CHUNK + k * 16, 16)]
                t16 = d16 - lo
                ok = jnp.logical_and(t16 >= 0, t16 < _HALF)
                rbuf[0, pl.ds(k * 16, 16)] = jnp.where(ok, t16, _HALF)
            pltpu.async_copy(xws_hbm.at[srcv.at[j]], rows, sem).wait()
            pltpu.sync_copy(rows, shagg.at[rbuf.at[0]], add=True)
            return carry

        lax.fori_loop(0, _CPW2, body, 0)
        plsc.subcore_barrier()
        pltpu.sync_copy(shagg.at[pl.ds(sid * _OPT, _OPT)],
                        out_hbm.at[cid, pl.ds(lo + sid * _OPT, _OPT)])
        plsc.subcore_barrier()


def _mm_body(x_ref, w_ref, o_ref):
    o_ref[...] = jnp.dot(x_ref[...], w_ref[...],
                         preferred_element_type=jnp.float32)


def _scale_body(xw_ref, hist_ref, xws_ref, norm_ref):
    ones = jnp.ones((_NS, 1), jnp.float32)
    deg = lax.dot_general(hist_ref[...], ones, (((0,), (0,)), ((), ())),
                          preferred_element_type=jnp.float32)
    norm = lax.rsqrt(jnp.maximum(deg, 1.0))
    norm_ref[...] = norm
    xws_ref[...] = xw_ref[...] * norm


def _loss_body(a_ref, norm_ref, b0_ref, wp_ref, bp_ref, o_ref):
    j = pl.program_id(0)
    agg = a_ref[0] + a_ref[1]
    h = jnp.maximum(agg * norm_ref[...] + b0_ref[...], 0.0)
    v = jnp.sum(wp_ref[...], axis=1, keepdims=True)
    c = jnp.sum(bp_ref[...])
    s = jnp.dot(h, v, preferred_element_type=jnp.float32) + c
    row = lax.broadcasted_iota(jnp.int32, (_BM, 1), 0) + j * _BM
    t = jnp.abs(s)
    term = t + 2.0 * jnp.log1p(jnp.exp(-t))
    term = jnp.where(row < _N, term, 0.0)
    part = (jnp.sum(term) / (2.0 * _N)).reshape(1, 1)

    @pl.when(j == 0)
    def _():
        o_ref[...] = jnp.zeros_like(part)

    o_ref[...] += part


def _tc_matmul(x, w):
    return pl.pallas_call(
        _mm_body,
        grid=(_NPAD // _BM,),
        in_specs=[pl.BlockSpec((_BM, _D), lambda i: (i, 0)),
                  pl.BlockSpec((_D, _D), lambda i: (0, 0))],
        out_specs=pl.BlockSpec((_BM, _D), lambda i: (i, 0)),
        out_shape=jax.ShapeDtypeStruct((_NPAD, _D), jnp.float32),
    )(x, w)


def _tc_scale(xw, hist):
    return pl.pallas_call(
        _scale_body,
        grid=(_NPAD // _BM,),
        in_specs=[pl.BlockSpec((_BM, _D), lambda i: (i, 0)),
                  pl.BlockSpec((_NS, _BM), lambda i: (0, i))],
        out_specs=[pl.BlockSpec((_BM, _D), lambda i: (i, 0)),
                   pl.BlockSpec((_BM, 1), lambda i: (i, 0))],
        out_shape=[jax.ShapeDtypeStruct((_NPAD, _D), jnp.float32),
                   jax.ShapeDtypeStruct((_NPAD, 1), jnp.float32)],
    )(xw, hist)


def _tc_loss(a, norm, b0r, wp, bpr):
    return pl.pallas_call(
        _loss_body,
        grid=(_NPAD // _BM,),
        in_specs=[pl.BlockSpec((_NC, _BM, _D), lambda i: (0, i, 0)),
                  pl.BlockSpec((_BM, 1), lambda i: (i, 0)),
                  pl.BlockSpec((1, _D), lambda i: (0, 0)),
                  pl.BlockSpec((_D, _D), lambda i: (0, 0)),
                  pl.BlockSpec((1, _D), lambda i: (0, 0))],
        out_specs=pl.BlockSpec((1, 1), lambda i: (0, 0)),
        out_shape=jax.ShapeDtypeStruct((1, 1), jnp.float32),
    )(a, norm, b0r, wp, bpr)


def kernel(features, edge_index, labels, W0, b0, Wp, bp):
    del labels
    xpad = jnp.pad(features, ((0, _NPAD - _N), (0, 0)))
    pad = _EPAD - _E
    dst_flat = jnp.concatenate(
        [edge_index[1], jnp.full((pad,), _N, jnp.int32)]).reshape(_NS, _EPW)
    pad2 = _EPAD2 - _E
    src_p = jnp.concatenate(
        [edge_index[0], jnp.zeros((pad2,), jnp.int32)]).reshape(
            _NC, _NS, _CPW2, _CHUNK)
    dst_p = jnp.concatenate(
        [edge_index[1], jnp.full((pad2,), _N, jnp.int32)]).reshape(
            _NC, _NS, _EPW2)
    zeros128 = jnp.zeros((_CHUNK, _D), jnp.float32)

    zeros_n = jnp.zeros((_NPAD,), jnp.float32)
    hist = _sc_deg(dst_flat, zeros_n)
    xw = _tc_matmul(xpad, W0)
    xws, norm = _tc_scale(xw, hist)
    agg = _sc_agg(xws, src_p, dst_p, zeros128)
    loss = _tc_loss(agg, norm,
                    b0.reshape(1, _D), Wp, bp.reshape(1, _D))
    return loss[0, 0]


# speedup vs baseline: 8.7132x; 1.3571x over previous
import functools

import jax
import jax.numpy as jnp
from jax import lax
from jax.experimental import pallas as pl
from jax.experimental.pallas import tpu as pltpu
from jax.experimental.pallas import tpu_sc as plsc

_N = 10000
_E = 320000
_D = 128

_NS = 16
_CHUNK = 128
_CPW = 158
_EPW = _CPW * _CHUNK
_EPAD = _NS * _EPW
_NPAD = 10240
_RPT = _NPAD // _NS
_BM = 2048

_mesh = plsc.VectorSubcoreMesh(
    core_axis_name="c", subcore_axis_name="s", num_cores=1)


@functools.partial(
    pl.kernel,
    out_type=jax.ShapeDtypeStruct((_NS, _NPAD), jnp.float32),
    mesh=_mesh,
    scratch_types=[
        pltpu.VMEM((_EPW,), jnp.int32),
        pltpu.VMEM((_NPAD,), jnp.float32),
    ],
    compiler_params=pltpu.CompilerParams(needs_layout_passes=False),
)
def _sc_deg(dst_hbm, zeros_hbm, out_hbm, dstv, hist):
    sid = lax.axis_index("s")
    pltpu.sync_copy(dst_hbm.at[sid], dstv)
    pltpu.sync_copy(zeros_hbm, hist)
    ones = jnp.full((16,), 1.0, jnp.float32)

    def body(i, carry):
        idx = dstv[pl.ds(i * 16, 16)]
        plsc.addupdate_scatter(hist, [idx], ones)
        return carry

    lax.fori_loop(0, _EPW // 16, body, 0)
    pltpu.sync_copy(hist, out_hbm.at[sid])


_NC = 2
_CPW_A = 101
_CPW_B = 56
_EA = _NS * _CPW_A * _CHUNK
_EB = _NS * _CPW_B * _CHUNK
_EPWM = _CPW_A * _CHUNK
_HALF = _NPAD // 2
_AROWS = _HALF + _CHUNK
_ZPT = _AROWS // _NS
_OPT = _HALF // _NS

_mesh2 = plsc.VectorSubcoreMesh(
    core_axis_name="c", subcore_axis_name="s", num_cores=_NC)


@functools.partial(
    pl.kernel,
    out_type=jax.ShapeDtypeStruct((_NC, _NPAD, _D), jnp.float32),
    mesh=_mesh2,
    scratch_types=[
        pltpu.VMEM((_CPW_A, _CHUNK), jnp.int32),
        pltpu.VMEM((_EPWM,), jnp.int32),
        pltpu.VMEM((1, _CHUNK), jnp.int32),
        pltpu.VMEM((_CHUNK, _D), jnp.float32),
        pltpu.VMEM((_CHUNK, _D), jnp.float32),
        pltpu.VMEM_SHARED((_AROWS, _D), jnp.float32),
        pltpu.SemaphoreType.DMA,
    ],
)
def _sc_agg(xws_hbm, src_hbm, dst_hbm, zeros_hbm, out_hbm,
            srcv, dstv, rbuf, rows, zbuf, shagg, sem):
    cid = lax.axis_index("c")
    sid = lax.axis_index("s")
    pltpu.sync_copy(src_hbm.at[cid, sid], srcv)
    pltpu.sync_copy(dst_hbm.at[cid, sid], dstv)
    pltpu.sync_copy(zeros_hbm, zbuf)

    def zero_acc():
        zb = sid * _ZPT
        off = 0
        while off < _ZPT:
            n = min(_CHUNK, _ZPT - off)
            pltpu.sync_copy(zbuf.at[pl.ds(0, n)],
                            shagg.at[pl.ds(zb + off, n)])
            off += n

    nb = jnp.where(cid == 0, _CPW_A, _CPW_B)
    for p in range(2):
        zero_acc()
        plsc.subcore_barrier()
        lo = p * _HALF

        def body(j, carry):
            for k in range(_CHUNK // 16):
                d16 = dstv[pl.ds(j * _CHUNK + k * 16, 16)]
                t16 = d16 - lo
                ok = jnp.logical_and(t16 >= 0, t16 < _HALF)
                rbuf[0, pl.ds(k * 16, 16)] = jnp.where(ok, t16, _HALF)
            pltpu.async_copy(xws_hbm.at[srcv.at[j]], rows, sem).wait()
            pltpu.sync_copy(rows, shagg.at[rbuf.at[0]], add=True)
            return carry

        lax.fori_loop(0, nb, body, 0)
        plsc.subcore_barrier()
        pltpu.sync_copy(shagg.at[pl.ds(sid * _OPT, _OPT)],
                        out_hbm.at[cid, pl.ds(lo + sid * _OPT, _OPT)])
        plsc.subcore_barrier()


def _mm_body(x_ref, w_ref, o_ref):
    o_ref[...] = jnp.dot(x_ref[...], w_ref[...],
                         preferred_element_type=jnp.float32)


def _scale_body(xw_ref, hist_ref, xws_ref, norm_ref):
    ones = jnp.ones((_NS, 1), jnp.float32)
    deg = lax.dot_general(hist_ref[...], ones, (((0,), (0,)), ((), ())),
                          preferred_element_type=jnp.float32)
    norm = lax.rsqrt(jnp.maximum(deg, 1.0))
    norm_ref[...] = norm
    xws_ref[...] = xw_ref[...] * norm


def _loss_body(a_ref, norm_ref, b0_ref, wp_ref, bp_ref, o_ref):
    j = pl.program_id(0)
    agg = a_ref[0] + a_ref[1]
    h = jnp.maximum(agg * norm_ref[...] + b0_ref[...], 0.0)
    v = jnp.sum(wp_ref[...], axis=1, keepdims=True)
    c = jnp.sum(bp_ref[...])
    s = jnp.dot(h, v, preferred_element_type=jnp.float32) + c
    row = lax.broadcasted_iota(jnp.int32, (_BM, 1), 0) + j * _BM
    t = jnp.abs(s)
    term = t + 2.0 * jnp.log1p(jnp.exp(-t))
    term = jnp.where(row < _N, term, 0.0)
    part = (jnp.sum(term) / (2.0 * _N)).reshape(1, 1)

    @pl.when(j == 0)
    def _():
        o_ref[...] = jnp.zeros_like(part)

    o_ref[...] += part


def _tc_matmul(x, w):
    return pl.pallas_call(
        _mm_body,
        grid=(_NPAD // _BM,),
        in_specs=[pl.BlockSpec((_BM, _D), lambda i: (i, 0)),
                  pl.BlockSpec((_D, _D), lambda i: (0, 0))],
        out_specs=pl.BlockSpec((_BM, _D), lambda i: (i, 0)),
        out_shape=jax.ShapeDtypeStruct((_NPAD, _D), jnp.float32),
    )(x, w)


def _tc_scale(xw, hist):
    return pl.pallas_call(
        _scale_body,
        grid=(_NPAD // _BM,),
        in_specs=[pl.BlockSpec((_BM, _D), lambda i: (i, 0)),
                  pl.BlockSpec((_NS, _BM), lambda i: (0, i))],
        out_specs=[pl.BlockSpec((_BM, _D), lambda i: (i, 0)),
                   pl.BlockSpec((_BM, 1), lambda i: (i, 0))],
        out_shape=[jax.ShapeDtypeStruct((_NPAD, _D), jnp.float32),
                   jax.ShapeDtypeStruct((_NPAD, 1), jnp.float32)],
    )(xw, hist)


def _tc_loss(a, norm, b0r, wp, bpr):
    return pl.pallas_call(
        _loss_body,
        grid=(_NPAD // _BM,),
        in_specs=[pl.BlockSpec((_NC, _BM, _D), lambda i: (0, i, 0)),
                  pl.BlockSpec((_BM, 1), lambda i: (i, 0)),
                  pl.BlockSpec((1, _D), lambda i: (0, 0)),
                  pl.BlockSpec((_D, _D), lambda i: (0, 0)),
                  pl.BlockSpec((1, _D), lambda i: (0, 0))],
        out_specs=pl.BlockSpec((1, 1), lambda i: (0, 0)),
        out_shape=jax.ShapeDtypeStruct((1, 1), jnp.float32),
    )(a, norm, b0r, wp, bpr)


def kernel(features, edge_index, labels, W0, b0, Wp, bp):
    del labels
    xpad = jnp.pad(features, ((0, _NPAD - _N), (0, 0)))
    pad = _EPAD - _E
    dst_flat = jnp.concatenate(
        [edge_index[1], jnp.full((pad,), _N, jnp.int32)]).reshape(_NS, _EPW)
    pad2 = _EA + _EB - _E
    src0 = edge_index[0, :_EA].reshape(_NS, _CPW_A, _CHUNK)
    src1 = jnp.concatenate(
        [edge_index[0, _EA:], jnp.zeros((pad2,), jnp.int32)]).reshape(
            _NS, _CPW_B, _CHUNK)
    src1 = jnp.pad(src1, ((0, 0), (0, _CPW_A - _CPW_B), (0, 0)))
    src_p = jnp.stack([src0, src1])
    dst0 = edge_index[1, :_EA].reshape(_NS, _CPW_A, _CHUNK)
    dst1 = jnp.concatenate(
        [edge_index[1, _EA:], jnp.full((pad2,), _N, jnp.int32)]).reshape(
            _NS, _CPW_B, _CHUNK)
    dst1 = jnp.pad(dst1, ((0, 0), (0, _CPW_A - _CPW_B), (0, 0)),
                   constant_values=_N)
    dst_p = jnp.stack([dst0, dst1]).reshape(_NC, _NS, _EPWM)
    zeros128 = jnp.zeros((_CHUNK, _D), jnp.float32)

    zeros_n = jnp.zeros((_NPAD,), jnp.float32)
    hist = _sc_deg(dst_flat, zeros_n)
    xw = _tc_matmul(xpad, W0)
    xws, norm = _tc_scale(xw, hist)
    agg = _sc_agg(xws, src_p, dst_p, zeros128)
    loss = _tc_loss(agg, norm,
                    b0.reshape(1, _D), Wp, bp.reshape(1, _D))
    return loss[0, 0]


# speedup vs baseline: 9.1473x; 1.0498x over previous
import functools

import jax
import jax.numpy as jnp
from jax import lax
from jax.experimental import pallas as pl
from jax.experimental.pallas import tpu as pltpu
from jax.experimental.pallas import tpu_sc as plsc

_N = 10000
_E = 320000
_D = 128

_NS = 16
_CHUNK = 128
_CPW = 158
_EPW = _CPW * _CHUNK
_EPAD = _NS * _EPW
_NPAD = 10240
_RPT = _NPAD // _NS
_BM = 2048

_mesh = plsc.VectorSubcoreMesh(
    core_axis_name="c", subcore_axis_name="s", num_cores=1)


@functools.partial(
    pl.kernel,
    out_type=jax.ShapeDtypeStruct((_NS, _NPAD), jnp.float32),
    mesh=_mesh,
    scratch_types=[
        pltpu.VMEM((_EPW,), jnp.int32),
        pltpu.VMEM((_NPAD,), jnp.float32),
    ],
    compiler_params=pltpu.CompilerParams(needs_layout_passes=False),
)
def _sc_deg(dst_hbm, zeros_hbm, out_hbm, dstv, hist):
    sid = lax.axis_index("s")
    pltpu.sync_copy(dst_hbm.at[sid], dstv)
    pltpu.sync_copy(zeros_hbm, hist)
    ones = jnp.full((16,), 1.0, jnp.float32)

    def body(i, carry):
        idx = dstv[pl.ds(i * 16, 16)]
        plsc.addupdate_scatter(hist, [idx], ones)
        return carry

    lax.fori_loop(0, _EPW // 16, body, 0)
    pltpu.sync_copy(hist, out_hbm.at[sid])


_NC = 2
_CPW_A = 95
_CPW_B = 62
_EA = _NS * _CPW_A * _CHUNK
_EB = _NS * _CPW_B * _CHUNK
_EPWM = _CPW_A * _CHUNK
_HALF = _NPAD // 2
_AROWS = _HALF + _CHUNK
_ZPT = _AROWS // _NS
_OPT = _HALF // _NS

_mesh2 = plsc.VectorSubcoreMesh(
    core_axis_name="c", subcore_axis_name="s", num_cores=_NC)


@functools.partial(
    pl.kernel,
    out_type=jax.ShapeDtypeStruct((_NC, _NPAD, _D), jnp.float32),
    mesh=_mesh2,
    scratch_types=[
        pltpu.VMEM((_CPW_A, _CHUNK), jnp.int32),
        pltpu.VMEM((_EPWM,), jnp.int32),
        pltpu.VMEM((1, _CHUNK), jnp.int32),
        pltpu.VMEM((_CHUNK, _D), jnp.float32),
        pltpu.VMEM((_CHUNK, _D), jnp.float32),
        pltpu.VMEM_SHARED((_AROWS, _D), jnp.float32),
        pltpu.SemaphoreType.DMA,
    ],
)
def _sc_agg(xws_hbm, src_hbm, dst_hbm, zeros_hbm, out_hbm,
            srcv, dstv, rbuf, rows, zbuf, shagg, sem):
    cid = lax.axis_index("c")
    sid = lax.axis_index("s")
    pltpu.sync_copy(src_hbm.at[cid, sid], srcv)
    pltpu.sync_copy(dst_hbm.at[cid, sid], dstv)
    pltpu.sync_copy(zeros_hbm, zbuf)

    def zero_acc():
        zb = sid * _ZPT
        off = 0
        while off < _ZPT:
            n = min(_CHUNK, _ZPT - off)
            pltpu.sync_copy(zbuf.at[pl.ds(0, n)],
                            shagg.at[pl.ds(zb + off, n)])
            off += n

    nb = jnp.where(cid == 0, _CPW_A, _CPW_B)
    for p in range(2):
        zero_acc()
        plsc.subcore_barrier()
        lo = p * _HALF

        def body(j, carry):
            for k in range(_CHUNK // 16):
                d16 = dstv[pl.ds(j * _CHUNK + k * 16, 16)]
                t16 = d16 - lo
                ok = jnp.logical_and(t16 >= 0, t16 < _HALF)
                rbuf[0, pl.ds(k * 16, 16)] = jnp.where(ok, t16, _HALF)
            pltpu.async_copy(xws_hbm.at[srcv.at[j]], rows, sem).wait()
            pltpu.sync_copy(rows, shagg.at[rbuf.at[0]], add=True)
            return carry

        lax.fori_loop(0, nb, body, 0)
        plsc.subcore_barrier()
        pltpu.sync_copy(shagg.at[pl.ds(sid * _OPT, _OPT)],
                        out_hbm.at[cid, pl.ds(lo + sid * _OPT, _OPT)])
        plsc.subcore_barrier()


def _mm_body(x_ref, w_ref, o_ref):
    o_ref[...] = jnp.dot(x_ref[...], w_ref[...],
                         preferred_element_type=jnp.float32)


def _scale_body(xw_ref, hist_ref, xws_ref, norm_ref):
    ones = jnp.ones((_NS, 1), jnp.float32)
    deg = lax.dot_general(hist_ref[...], ones, (((0,), (0,)), ((), ())),
                          preferred_element_type=jnp.float32)
    norm = lax.rsqrt(jnp.maximum(deg, 1.0))
    norm_ref[...] = norm
    xws_ref[...] = xw_ref[...] * norm


def _loss_body(a_ref, norm_ref, b0_ref, wp_ref, bp_ref, o_ref):
    j = pl.program_id(0)
    agg = a_ref[0] + a_ref[1]
    h = jnp.maximum(agg * norm_ref[...] + b0_ref[...], 0.0)
    v = jnp.sum(wp_ref[...], axis=1, keepdims=True)
    c = jnp.sum(bp_ref[...])
    s = jnp.dot(h, v, preferred_element_type=jnp.float32) + c
    row = lax.broadcasted_iota(jnp.int32, (_BM, 1), 0) + j * _BM
    t = jnp.abs(s)
    term = t + 2.0 * jnp.log1p(jnp.exp(-t))
    term = jnp.where(row < _N, term, 0.0)
    part = (jnp.sum(term) / (2.0 * _N)).reshape(1, 1)

    @pl.when(j == 0)
    def _():
        o_ref[...] = jnp.zeros_like(part)

    o_ref[...] += part


def _tc_matmul(x, w):
    return pl.pallas_call(
        _mm_body,
        grid=(_NPAD // _BM,),
        in_specs=[pl.BlockSpec((_BM, _D), lambda i: (i, 0)),
                  pl.BlockSpec((_D, _D), lambda i: (0, 0))],
        out_specs=pl.BlockSpec((_BM, _D), lambda i: (i, 0)),
        out_shape=jax.ShapeDtypeStruct((_NPAD, _D), jnp.float32),
    )(x, w)


def _tc_scale(xw, hist):
    return pl.pallas_call(
        _scale_body,
        grid=(_NPAD // _BM,),
        in_specs=[pl.BlockSpec((_BM, _D), lambda i: (i, 0)),
                  pl.BlockSpec((_NS, _BM), lambda i: (0, i))],
        out_specs=[pl.BlockSpec((_BM, _D), lambda i: (i, 0)),
                   pl.BlockSpec((_BM, 1), lambda i: (i, 0))],
        out_shape=[jax.ShapeDtypeStruct((_NPAD, _D), jnp.float32),
                   jax.ShapeDtypeStruct((_NPAD, 1), jnp.float32)],
    )(xw, hist)


def _tc_loss(a, norm, b0r, wp, bpr):
    return pl.pallas_call(
        _loss_body,
        grid=(_NPAD // _BM,),
        in_specs=[pl.BlockSpec((_NC, _BM, _D), lambda i: (0, i, 0)),
                  pl.BlockSpec((_BM, 1), lambda i: (i, 0)),
                  pl.BlockSpec((1, _D), lambda i: (0, 0)),
                  pl.BlockSpec((_D, _D), lambda i: (0, 0)),
                  pl.BlockSpec((1, _D), lambda i: (0, 0))],
        out_specs=pl.BlockSpec((1, 1), lambda i: (0, 0)),
        out_shape=jax.ShapeDtypeStruct((1, 1), jnp.float32),
    )(a, norm, b0r, wp, bpr)


def kernel(features, edge_index, labels, W0, b0, Wp, bp):
    del labels
    xpad = jnp.pad(features, ((0, _NPAD - _N), (0, 0)))
    pad = _EPAD - _E
    dst_flat = jnp.concatenate(
        [edge_index[1], jnp.full((pad,), _N, jnp.int32)]).reshape(_NS, _EPW)
    pad2 = _EA + _EB - _E
    src0 = edge_index[0, :_EA].reshape(_NS, _CPW_A, _CHUNK)
    src1 = jnp.concatenate(
        [edge_index[0, _EA:], jnp.zeros((pad2,), jnp.int32)]).reshape(
            _NS, _CPW_B, _CHUNK)
    src1 = jnp.pad(src1, ((0, 0), (0, _CPW_A - _CPW_B), (0, 0)))
    src_p = jnp.stack([src0, src1])
    dst0 = edge_index[1, :_EA].reshape(_NS, _CPW_A, _CHUNK)
    dst1 = jnp.concatenate(
        [edge_index[1, _EA:], jnp.full((pad2,), _N, jnp.int32)]).reshape(
            _NS, _CPW_B, _CHUNK)
    dst1 = jnp.pad(dst1, ((0, 0), (0, _CPW_A - _CPW_B), (0, 0)),
                   constant_values=_N)
    dst_p = jnp.stack([dst0, dst1]).reshape(_NC, _NS, _EPWM)
    zeros128 = jnp.zeros((_CHUNK, _D), jnp.float32)

    zeros_n = jnp.zeros((_NPAD,), jnp.float32)
    hist = _sc_deg(dst_flat, zeros_n)
    xw = _tc_matmul(xpad, W0)
    xws, norm = _tc_scale(xw, hist)
    agg = _sc_agg(xws, src_p, dst_p, zeros128)
    loss = _tc_loss(agg, norm,
                    b0.reshape(1, _D), Wp, bp.reshape(1, _D))
    return loss[0, 0]
